# Initial kernel scaffold; baseline (speedup 1.0000x reference)
import functools

import jax
import jax.numpy as jnp
from jax import lax
from jax.experimental import pallas as pl
from jax.experimental.pallas import tpu as pltpu
from jax.experimental.pallas import tpu_sc as plsc

N = 10000
K = 32
D = 128
E = N * K
L = 16
C = D // L
NB = 4
NUM_BATCHES = N // NB
NW = 32
BATCHES_PER_W = -(-NUM_BATCHES // NW)
BR = 1000


def _tc_linear(feature, W_u, b_u, W_v, b_v):
    def body(x_ref, wu_ref, bu_ref, wv_ref, bv_ref, u_ref, v_ref):
        x = x_ref[...]
        u_ref[...] = lax.dot_general(
            x, wu_ref[...], (((1,), (1,)), ((), ())),
            preferred_element_type=jnp.float32) + bu_ref[...]
        v_ref[...] = lax.dot_general(
            x, wv_ref[...], (((1,), (1,)), ((), ())),
            preferred_element_type=jnp.float32) + bv_ref[...]

    return pl.pallas_call(
        body,
        grid=(N // BR,),
        in_specs=[
            pl.BlockSpec((BR, D), lambda i: (i, 0)),
            pl.BlockSpec((D, D), lambda i: (0, 0)),
            pl.BlockSpec((1, D), lambda i: (0, 0)),
            pl.BlockSpec((D, D), lambda i: (0, 0)),
            pl.BlockSpec((1, D), lambda i: (0, 0)),
        ],
        out_specs=[
            pl.BlockSpec((BR, D), lambda i: (i, 0)),
            pl.BlockSpec((BR, D), lambda i: (i, 0)),
        ],
        out_shape=[
            jax.ShapeDtypeStruct((N, D), jnp.float32),
            jax.ShapeDtypeStruct((N, D), jnp.float32),
        ],
    )(feature, W_u, b_u.reshape(1, D), W_v, b_v.reshape(1, D))


def _sc_edges(u_feat, v_feat, src, sp, wvec, bvec):
    mesh = plsc.VectorSubcoreMesh(core_axis_name="c", subcore_axis_name="s")

    @functools.partial(
        pl.kernel,
        out_type=[
            jax.ShapeDtypeStruct((N, K), jnp.float32),
            jax.ShapeDtypeStruct((N, D), jnp.float32),
        ],
        mesh=mesh,
        scratch_types=[
            pltpu.VMEM((NB * K,), jnp.int32),
            pltpu.VMEM((NB * K,), jnp.float32),
            pltpu.VMEM((NB, D), jnp.float32),
            pltpu.VMEM((NB * K, D), jnp.float32),
            pltpu.VMEM((K, L), jnp.float32),
            pltpu.VMEM((D,), jnp.float32),
            pltpu.VMEM((L,), jnp.float32),
            pltpu.VMEM((NB, K), jnp.float32),
            pltpu.VMEM((NB, D), jnp.float32),
            pltpu.SemaphoreType.DMA,
        ],
    )
    def k(u_hbm, v_hbm, src_hbm, sp_hbm, w_hbm, b_hbm, pos_hbm, str_hbm,
          idx_v, sp_v, vrows, rows, parts, w_v, b_v, pos_s, str_s, sem):
        wid = lax.axis_index("s") * 2 + lax.axis_index("c")
        pltpu.sync_copy(w_hbm, w_v)
        pltpu.sync_copy(b_hbm, b_v)
        iota = lax.iota(jnp.int32, L)

        def batch_body(i, carry):
            b = wid + i * NW

            @pl.when(b < NUM_BATCHES)
            def _():
                nb = b * NB
                pltpu.sync_copy(src_hbm.at[pl.ds(nb * K, NB * K)], idx_v)
                pltpu.sync_copy(sp_hbm.at[pl.ds(nb * K, NB * K)], sp_v)
                pltpu.sync_copy(v_hbm.at[pl.ds(nb, NB)], vrows)
                pltpu.async_copy(u_hbm.at[idx_v], rows, sem).wait()
                wch = [w_v[pl.ds(c * L, L)] for c in range(C)]
                for j in range(NB):
                    vch = [vrows[j, pl.ds(c * L, L)] for c in range(C)]

                    def edge(kk, saccs):
                        e = j * K + kk
                        s = sp_v[e]
                        m = [jnp.maximum(vch[c] + s * rows[e, pl.ds(c * L, L)],
                                         0.0) for c in range(C)]
                        p0 = m[0] * wch[0]
                        p1 = m[1] * wch[1]
                        for c in range(2, C, 2):
                            p0 = p0 + m[c] * wch[c]
                        for c in range(3, C, 2):
                            p1 = p1 + m[c] * wch[c]
                        parts[kk, :] = p0 + p1
                        return tuple(saccs[c] + m[c] for c in range(C))

                    saccs = lax.fori_loop(
                        0, K, edge,
                        tuple(jnp.zeros((L,), jnp.float32) for _ in range(C)))
                    for c in range(C):
                        str_s[j, pl.ds(c * L, L)] = saccs[c] * (1.0 / K)
                    for g in range(K // L):
                        r = b_v[...]
                        rowi = iota + g * L
                        for l in range(L):
                            coli = jnp.full((L,), l, jnp.int32)
                            r = r + plsc.load_gather(parts, [rowi, coli])
                        pos_s[j, pl.ds(g * L, L)] = r
                pltpu.sync_copy(pos_s, pos_hbm.at[pl.ds(nb, NB)])
                pltpu.sync_copy(str_s, str_hbm.at[pl.ds(nb, NB)])

            return carry

        lax.fori_loop(0, BATCHES_PER_W, batch_body, 0)

    return k(u_feat, v_feat, src, sp, wvec, bvec)


def kernel(feature, sp_dist, dists_max, edge_src, edge_dst, anchor_eid,
           W_u, b_u, W_v, b_v, W_out, b_out):
    u_feat, v_feat = _tc_linear(feature, W_u, b_u, W_v, b_v)
    src = edge_src.astype(jnp.int32)
    wvec = W_out.reshape(D)
    bvec = jnp.broadcast_to(b_out.astype(jnp.float32), (L,))
    out_position, out_structure = _sc_edges(
        u_feat, v_feat, src, sp_dist, wvec, bvec)
    return (out_position, out_structure)


# speedup vs baseline: 4.3180x; 4.3180x over previous
import functools

import jax
import jax.numpy as jnp
from jax import lax
from jax.experimental import pallas as pl
from jax.experimental.pallas import tpu as pltpu
from jax.experimental.pallas import tpu_sc as plsc

N = 10000
K = 32
D = 128
E = N * K
L = 16
C = D // L
NB = 4
NUM_BATCHES = N // NB
NW = 32
BATCHES_PER_W = -(-NUM_BATCHES // NW)
BR = 1000


def _tc_linear(feature, W_u, b_u, W_v, b_v):
    def body(x_ref, wu_ref, bu_ref, wv_ref, bv_ref, u_ref, v_ref):
        x = x_ref[...]
        u_ref[...] = lax.dot_general(
            x, wu_ref[...], (((1,), (1,)), ((), ())),
            preferred_element_type=jnp.float32) + bu_ref[...]
        v_ref[...] = lax.dot_general(
            x, wv_ref[...], (((1,), (1,)), ((), ())),
            preferred_element_type=jnp.float32) + bv_ref[...]

    return pl.pallas_call(
        body,
        grid=(N // BR,),
        in_specs=[
            pl.BlockSpec((BR, D), lambda i: (i, 0)),
            pl.BlockSpec((D, D), lambda i: (0, 0)),
            pl.BlockSpec((1, D), lambda i: (0, 0)),
            pl.BlockSpec((D, D), lambda i: (0, 0)),
            pl.BlockSpec((1, D), lambda i: (0, 0)),
        ],
        out_specs=[
            pl.BlockSpec((BR, D), lambda i: (i, 0)),
            pl.BlockSpec((BR, D), lambda i: (i, 0)),
        ],
        out_shape=[
            jax.ShapeDtypeStruct((N, D), jnp.float32),
            jax.ShapeDtypeStruct((N, D), jnp.float32),
        ],
    )(feature, W_u, b_u.reshape(1, D), W_v, b_v.reshape(1, D))


def _sc_edges(u_feat, v_feat, src, sp, wvec, bvec):
    mesh = plsc.VectorSubcoreMesh(core_axis_name="c", subcore_axis_name="s")

    @functools.partial(
        pl.kernel,
        out_type=[
            jax.ShapeDtypeStruct((N, K), jnp.float32),
            jax.ShapeDtypeStruct((N, D), jnp.float32),
        ],
        mesh=mesh,
        compiler_params=pltpu.CompilerParams(needs_layout_passes=False),
        scratch_types=[
            pltpu.VMEM((NB * K,), jnp.int32),
            pltpu.VMEM((NB * K,), jnp.float32),
            pltpu.VMEM((NB, D), jnp.float32),
            pltpu.VMEM((NB * K, D), jnp.float32),
            pltpu.VMEM((K, L), jnp.float32),
            pltpu.VMEM((D,), jnp.float32),
            pltpu.VMEM((L,), jnp.float32),
            pltpu.VMEM((NB, K), jnp.float32),
            pltpu.VMEM((NB, D), jnp.float32),
            pltpu.SemaphoreType.DMA,
        ],
    )
    def k(u_hbm, v_hbm, src_hbm, sp_hbm, w_hbm, b_hbm, pos_hbm, str_hbm,
          idx_v, sp_v, vrows, rows, parts, w_v, b_v, pos_s, str_s, sem):
        wid = lax.axis_index("s") * 2 + lax.axis_index("c")
        pltpu.sync_copy(w_hbm, w_v)
        pltpu.sync_copy(b_hbm, b_v)
        iota = lax.iota(jnp.int32, L)

        def batch_body(i, carry):
            b = wid + i * NW

            @pl.when(b < NUM_BATCHES)
            def _():
                nb = b * NB
                pltpu.sync_copy(src_hbm.at[pl.ds(nb * K, NB * K)], idx_v)
                pltpu.sync_copy(sp_hbm.at[pl.ds(nb * K, NB * K)], sp_v)
                pltpu.sync_copy(v_hbm.at[pl.ds(nb, NB)], vrows)
                pltpu.async_copy(u_hbm.at[idx_v], rows, sem).wait()
                wch = [w_v[pl.ds(c * L, L)] for c in range(C)]
                for j in range(NB):
                    vch = [vrows[j, pl.ds(c * L, L)] for c in range(C)]

                    def edge(kk, saccs):
                        e = j * K + kk
                        s = plsc.load_gather(sp_v, [jnp.full((L,), 0, jnp.int32) + e])
                        m = [jnp.maximum(vch[c] + s * rows[e, pl.ds(c * L, L)],
                                         0.0) for c in range(C)]
                        p0 = m[0] * wch[0]
                        p1 = m[1] * wch[1]
                        for c in range(2, C, 2):
                            p0 = p0 + m[c] * wch[c]
                        for c in range(3, C, 2):
                            p1 = p1 + m[c] * wch[c]
                        parts[kk, :] = p0 + p1
                        return tuple(saccs[c] + m[c] for c in range(C))

                    saccs = lax.fori_loop(
                        0, K, edge,
                        tuple(jnp.zeros((L,), jnp.float32) for _ in range(C)))
                    for c in range(C):
                        str_s[j, pl.ds(c * L, L)] = saccs[c] * (1.0 / K)
                    for g in range(K // L):
                        r = b_v[...]
                        rowi = iota + g * L
                        for l in range(L):
                            coli = jnp.full((L,), l, jnp.int32)
                            r = r + plsc.load_gather(parts, [rowi, coli])
                        pos_s[j, pl.ds(g * L, L)] = r
                pltpu.sync_copy(pos_s, pos_hbm.at[pl.ds(nb, NB)])
                pltpu.sync_copy(str_s, str_hbm.at[pl.ds(nb, NB)])

            return carry

        lax.fori_loop(0, BATCHES_PER_W, batch_body, 0)

    return k(u_feat, v_feat, src, sp, wvec, bvec)


def kernel(feature, sp_dist, dists_max, edge_src, edge_dst, anchor_eid,
           W_u, b_u, W_v, b_v, W_out, b_out):
    u_feat, v_feat = _tc_linear(feature, W_u, b_u, W_v, b_v)
    src = edge_src.astype(jnp.int32)
    wvec = W_out.reshape(D)
    bvec = jnp.broadcast_to(b_out.astype(jnp.float32), (L,))
    out_position, out_structure = _sc_edges(
        u_feat, v_feat, src, sp_dist, wvec, bvec)
    return (out_position, out_structure)


# speedup vs baseline: 6.8083x; 1.5767x over previous
import functools

import jax
import jax.numpy as jnp
from jax import lax
from jax.experimental import pallas as pl
from jax.experimental.pallas import tpu as pltpu
from jax.experimental.pallas import tpu_sc as plsc

N = 10000
K = 32
D = 128
E = N * K
L = 16
C = D // L
NB = 4
NUM_BATCHES = N // NB
NW = 32
BATCHES_PER_W = -(-NUM_BATCHES // NW)
BR = 1000


def _tc_linear(feature, W_u, b_u, W_v, b_v):
    def body(x_ref, wu_ref, bu_ref, wv_ref, bv_ref, u_ref, v_ref):
        x = x_ref[...]
        u_ref[...] = lax.dot_general(
            x, wu_ref[...], (((1,), (1,)), ((), ())),
            preferred_element_type=jnp.float32) + bu_ref[...]
        v_ref[...] = lax.dot_general(
            x, wv_ref[...], (((1,), (1,)), ((), ())),
            preferred_element_type=jnp.float32) + bv_ref[...]

    return pl.pallas_call(
        body,
        grid=(N // BR,),
        in_specs=[
            pl.BlockSpec((BR, D), lambda i: (i, 0)),
            pl.BlockSpec((D, D), lambda i: (0, 0)),
            pl.BlockSpec((1, D), lambda i: (0, 0)),
            pl.BlockSpec((D, D), lambda i: (0, 0)),
            pl.BlockSpec((1, D), lambda i: (0, 0)),
        ],
        out_specs=[
            pl.BlockSpec((BR, D), lambda i: (i, 0)),
            pl.BlockSpec((BR, D), lambda i: (i, 0)),
        ],
        out_shape=[
            jax.ShapeDtypeStruct((N, D), jnp.float32),
            jax.ShapeDtypeStruct((N, D), jnp.float32),
        ],
    )(feature, W_u, b_u.reshape(1, D), W_v, b_v.reshape(1, D))


def _sc_edges(u_feat, v_feat, src, sp, wvec, bvec):
    mesh = plsc.VectorSubcoreMesh(core_axis_name="c", subcore_axis_name="s")
    PAIRS = (BATCHES_PER_W + 1) // 2

    @functools.partial(
        pl.kernel,
        out_type=[
            jax.ShapeDtypeStruct((N, K), jnp.float32),
            jax.ShapeDtypeStruct((N, D), jnp.float32),
        ],
        mesh=mesh,
        compiler_params=pltpu.CompilerParams(needs_layout_passes=False),
        scratch_types=[
            pltpu.VMEM((NB * K,), jnp.int32),
            pltpu.VMEM((NB * K,), jnp.int32),
            pltpu.VMEM((NB * K,), jnp.float32),
            pltpu.VMEM((NB * K,), jnp.float32),
            pltpu.VMEM((NB, D), jnp.float32),
            pltpu.VMEM((NB, D), jnp.float32),
            pltpu.VMEM((NB * K, D), jnp.float32),
            pltpu.VMEM((NB * K, D), jnp.float32),
            pltpu.VMEM((K, L), jnp.float32),
            pltpu.VMEM((D,), jnp.float32),
            pltpu.VMEM((L,), jnp.float32),
            pltpu.VMEM((NB, K), jnp.float32),
            pltpu.VMEM((NB, K), jnp.float32),
            pltpu.VMEM((NB, D), jnp.float32),
            pltpu.VMEM((NB, D), jnp.float32),
            pltpu.SemaphoreType.DMA,
            pltpu.SemaphoreType.DMA,
            pltpu.SemaphoreType.DMA,
            pltpu.SemaphoreType.DMA,
            pltpu.SemaphoreType.DMA,
            pltpu.SemaphoreType.DMA,
        ],
    )
    def k(u_hbm, v_hbm, src_hbm, sp_hbm, w_hbm, b_hbm, pos_hbm, str_hbm,
          idx_v0, idx_v1, sp_v0, sp_v1, vr0, vr1, rows0, rows1,
          parts, w_v, b_v, pos_s0, pos_s1, str_s0, str_s1,
          sem_a0, sem_a1, sem_g0, sem_g1, sem_o0, sem_o1):
        idx_v = (idx_v0, idx_v1)
        sp_v = (sp_v0, sp_v1)
        vrows = (vr0, vr1)
        rows = (rows0, rows1)
        pos_s = (pos_s0, pos_s1)
        str_s = (str_s0, str_s1)
        sem_a = (sem_a0, sem_a1)
        sem_g = (sem_g0, sem_g1)
        sem_o = (sem_o0, sem_o1)

        wid = lax.axis_index("s") * 2 + lax.axis_index("c")
        pltpu.sync_copy(w_hbm, w_v)
        pltpu.sync_copy(b_hbm, b_v)
        iota = lax.iota(jnp.int32, L)

        def start_aux(b, p):
            nb = b * NB
            pltpu.async_copy(src_hbm.at[pl.ds(nb * K, NB * K)], idx_v[p], sem_a[p])
            pltpu.async_copy(sp_hbm.at[pl.ds(nb * K, NB * K)], sp_v[p], sem_a[p])
            pltpu.async_copy(v_hbm.at[pl.ds(nb, NB)], vrows[p], sem_a[p])

        def wait_aux(p):
            pltpu.make_async_copy(src_hbm.at[pl.ds(0, NB * K)], idx_v[p], sem_a[p]).wait()
            pltpu.make_async_copy(sp_hbm.at[pl.ds(0, NB * K)], sp_v[p], sem_a[p]).wait()
            pltpu.make_async_copy(v_hbm.at[pl.ds(0, NB)], vrows[p], sem_a[p]).wait()

        def start_gather(p):
            pltpu.async_copy(u_hbm.at[idx_v[p]], rows[p], sem_g[p])

        def wait_gather(p):
            pltpu.make_async_copy(u_hbm.at[idx_v[p]], rows[p], sem_g[p]).wait()

        def start_out(b, p):
            nb = b * NB
            pltpu.async_copy(pos_s[p], pos_hbm.at[pl.ds(nb, NB)], sem_o[p])
            pltpu.async_copy(str_s[p], str_hbm.at[pl.ds(nb, NB)], sem_o[p])

        def wait_out(p):
            pltpu.make_async_copy(pos_s[p], pos_hbm.at[pl.ds(0, NB)], sem_o[p]).wait()
            pltpu.make_async_copy(str_s[p], str_hbm.at[pl.ds(0, NB)], sem_o[p]).wait()

        def compute(p):
            wch = [w_v[pl.ds(c * L, L)] for c in range(C)]
            for j in range(NB):
                vch = [vrows[p][j, pl.ds(c * L, L)] for c in range(C)]

                def edge(kk, saccs):
                    e = j * K + kk
                    s = plsc.load_gather(
                        sp_v[p], [jnp.full((L,), 0, jnp.int32) + e])
                    m = [jnp.maximum(vch[c] + s * rows[p][e, pl.ds(c * L, L)],
                                     0.0) for c in range(C)]
                    p0 = m[0] * wch[0]
                    p1 = m[1] * wch[1]
                    for c in range(2, C, 2):
                        p0 = p0 + m[c] * wch[c]
                    for c in range(3, C, 2):
                        p1 = p1 + m[c] * wch[c]
                    parts[kk, :] = p0 + p1
                    return tuple(saccs[c] + m[c] for c in range(C))

                saccs = lax.fori_loop(
                    0, K, edge,
                    tuple(jnp.zeros((L,), jnp.float32) for _ in range(C)))
                for c in range(C):
                    str_s[p][j, pl.ds(c * L, L)] = saccs[c] * (1.0 / K)
                for g in range(K // L):
                    r = b_v[...]
                    rowi = iota + g * L
                    for l in range(L):
                        coli = jnp.full((L,), l, jnp.int32)
                        r = r + plsc.load_gather(parts, [rowi, coli])
                    pos_s[p][j, pl.ds(g * L, L)] = r

        start_aux(wid, 0)
        wait_aux(0)
        start_gather(0)
        start_aux(wid + NW, 1)

        def pair_body(ip, carry):
            for par in (0, 1):
                i = ip * 2 + par
                b = wid + i * NW

                @pl.when(b < NUM_BATCHES)
                def _():
                    wait_gather(par)

                    @pl.when(b + NW < NUM_BATCHES)
                    def _():
                        wait_aux(1 - par)
                        start_gather(1 - par)

                    @pl.when(ip >= 1)
                    def _():
                        wait_out(par)

                    compute(par)
                    start_out(b, par)

                    @pl.when(b + 2 * NW < NUM_BATCHES)
                    def _():
                        start_aux(b + 2 * NW, par)

            return carry

        lax.fori_loop(0, PAIRS, pair_body, 0)
        wait_out(0)
        wait_out(1)

    return k(u_feat, v_feat, src, sp, wvec, bvec)


def kernel(feature, sp_dist, dists_max, edge_src, edge_dst, anchor_eid,
           W_u, b_u, W_v, b_v, W_out, b_out):
    u_feat, v_feat = _tc_linear(feature, W_u, b_u, W_v, b_v)
    src = edge_src.astype(jnp.int32)
    wvec = W_out.reshape(D)
    bvec = jnp.broadcast_to(b_out.astype(jnp.float32), (L,))
    out_position, out_structure = _sc_edges(
        u_feat, v_feat, src, sp_dist, wvec, bvec)
    return (out_position, out_structure)
